# Initial kernel scaffold; baseline (speedup 1.0000x reference)
import jax
import jax.numpy as jnp
from jax.experimental import pallas as pl


def kernel(baselines, frames, gains):
    raise NotImplementedError("write your pallas kernel here")


# speedup vs baseline: 255.3273x; 255.3273x over previous
import jax
import jax.numpy as jnp
from jax import lax
from jax.experimental import pallas as pl
from jax.experimental.pallas import tpu as pltpu
from jax.experimental.pallas import tpu_sc as plsc

NSITES = 64
NTIMES = 8192
NBASE = 2016
NFRAMES = 4096

_PI = 3.141592653589793
_TWO_PI = 6.283185307179586

L = 16
NC = 2
NS = 16
NW = NC * NS
FPW = NFRAMES // NW
CH = 8
NCHUNK = FPW // CH
ROW = 2 * NBASE
CVECS = NBASE // L


def _wrap(x):
    r = lax.rem(x + _PI, _TWO_PI)
    r = jnp.where(r < 0.0, r + _TWO_PI, r)
    return r - _PI


def _sc_body(bl_hbm, frames_hbm, gt_hbm, gi_hbm, gj_hbm,
             fidx_v, g2_v, gflat_v, rows_v, oi_v, oj_v, sem):
    wid = lax.axis_index("s") * NC + lax.axis_index("c")
    base = wid * FPW
    iota = lax.iota(jnp.int32, L)
    perm = lax.bitwise_and(iota * 2, L - 1)
    permj = perm + 1
    lower = iota < (L // 2)

    pltpu.sync_copy(frames_hbm.at[pl.ds(base, FPW)], fidx_v)
    pltpu.async_copy(gt_hbm.at[fidx_v], g2_v, sem).wait()

    def clip_body(k, carry):
        r = lax.shift_right_logical(k, 2)
        c = lax.bitwise_and(k, 3) * L
        x = g2_v[r, pl.ds(c, L)]
        gflat_v[pl.ds(k * L, L)] = _wrap(x)
        return carry

    lax.fori_loop(0, FPW * NSITES // L, clip_body, 0)

    def chunk_body(ch, carry):
        pltpu.async_copy(
            bl_hbm.at[fidx_v.at[pl.ds(ch * CH, CH)]], rows_v, sem).wait()

        def frame_body(f, carry2):
            lf = ch * CH + f
            fofs = jnp.full((L,), lf * NSITES, jnp.int32)
            obase = f * NBASE

            def c_body(c, carry3):
                a = rows_v[f, pl.ds(c * 2 * L, L)]
                b = rows_v[f, pl.ds(c * 2 * L + L, L)]
                va = plsc.load_gather(gflat_v, [a + fofs])
                vb = plsc.load_gather(gflat_v, [b + fofs])
                gia = va.at[perm].get(mode="promise_in_bounds")
                gib = vb.at[perm].get(mode="promise_in_bounds")
                gja = va.at[permj].get(mode="promise_in_bounds")
                gjb = vb.at[permj].get(mode="promise_in_bounds")
                oi_v[pl.ds(obase + c * L, L)] = jnp.where(lower, gia, gib)
                oj_v[pl.ds(obase + c * L, L)] = jnp.where(lower, gja, gjb)
                return carry3

            lax.fori_loop(0, CVECS, c_body, 0)
            return carry2

        lax.fori_loop(0, CH, frame_body, 0)

        off = (base + ch * CH) * NBASE
        pltpu.sync_copy(oi_v, gi_hbm.at[pl.ds(off, CH * NBASE)])
        pltpu.sync_copy(oj_v, gj_hbm.at[pl.ds(off, CH * NBASE)])
        return carry

    lax.fori_loop(0, NCHUNK, chunk_body, 0)


def _phase_gains_sc(bl2, frames, gt):
    k = pl.kernel(
        _sc_body,
        out_type=[
            jax.ShapeDtypeStruct((NFRAMES * NBASE,), jnp.float32),
            jax.ShapeDtypeStruct((NFRAMES * NBASE,), jnp.float32),
        ],
        mesh=plsc.VectorSubcoreMesh(core_axis_name="c", subcore_axis_name="s"),
        scratch_types=[
            pltpu.VMEM((FPW,), jnp.int32),
            pltpu.VMEM((FPW, NSITES), jnp.float32),
            pltpu.VMEM((FPW * NSITES,), jnp.float32),
            pltpu.VMEM((CH, ROW), jnp.int32),
            pltpu.VMEM((CH * NBASE,), jnp.float32),
            pltpu.VMEM((CH * NBASE,), jnp.float32),
            pltpu.SemaphoreType.DMA,
        ],
        compiler_params=pltpu.CompilerParams(
            needs_layout_passes=False, use_tc_tiling_on_sc=False),
    )
    return k(bl2, frames, gt)


def kernel(baselines, frames, gains):
    bl2 = baselines.reshape(NTIMES, ROW)
    gt = gains.T
    gi, gj = _phase_gains_sc(bl2, frames, gt)
    return gi.reshape(NFRAMES, NBASE), gj.reshape(NFRAMES, NBASE)
